# Initial kernel scaffold; baseline (speedup 1.0000x reference)
import jax
import jax.numpy as jnp
from jax.experimental import pallas as pl


def kernel(x, labels, W, b):
    raise NotImplementedError("write your pallas kernel here")


# speedup vs baseline: 4.0923x; 4.0923x over previous
import functools

import jax
import jax.numpy as jnp
from jax import lax
from jax.experimental import pallas as pl
from jax.experimental.pallas import tpu as pltpu
from jax.experimental.pallas import tpu_sc as plsc

K_SEL = 16384
B_, C_, H_, W_ = 64, 3, 224, 224
CHAN = H_ * W_
D_ = C_ * CHAN
HALF = D_ // 2
NC_ = 100
DT = 1024
IDX_CHUNK = 2048


def _negabs_grad_body(lab_ref, w_ref, o_ref):
    g = lax.dot_general(
        lab_ref[...], w_ref[...], (((1,), (1,)), ((), ())),
        preferred_element_type=jnp.float32)
    o_ref[...] = -jnp.abs(g)


def _minmax_body(x_ref, o_ref):
    row = x_ref[...].reshape(1, D_)
    pos = lax.broadcasted_iota(jnp.int32, (1, D_), 1)
    p = pos % CHAN
    ch = pos // CHAN
    valid = p < (CHAN - 1)
    lane = lax.broadcasted_iota(jnp.int32, (1, 16), 1)
    r = jnp.zeros((1, 16), jnp.float32)
    for c in range(C_):
        sel = valid & (ch == c)
        mn = jnp.min(jnp.where(sel, row, jnp.inf))
        mx = jnp.max(jnp.where(sel, row, -jnp.inf))
        r = jnp.where(lane == c, mn, r)
        r = jnp.where(lane == c + 8, mx, r)
    o_ref[...] = r.reshape(1, 1, 16)


def _sc_scatter_body(x_hbm, idx_hbm, u_hbm, mm_hbm, out_hbm,
                     half_v, idx_v, u_v, mm_v):
    cid = lax.axis_index("c")
    sid = lax.axis_index("s")
    wid = sid * 2 + cid
    for i in range(4):
        h = i // 2
        b = wid + 32 * (i % 2)
        pltpu.sync_copy(x_hbm.at[b, pl.ds(h * HALF, HALF)], half_v)
        pltpu.sync_copy(mm_hbm.at[b], mm_v)
        for cch in range(K_SEL // IDX_CHUNK):
            pltpu.sync_copy(idx_hbm.at[b, pl.ds(cch * IDX_CHUNK, IDX_CHUNK)],
                            idx_v)
            pltpu.sync_copy(u_hbm.at[b, pl.ds(cch * IDX_CHUNK, IDX_CHUNK)],
                            u_v)

            def inner(j, carry):
                iv = idx_v[pl.ds(j * 16, 16)]
                uv = u_v[pl.ds(j * 16, 16)]
                chv = lax.div(iv, CHAN)
                mn = plsc.load_gather(mm_v, [chv])
                mx = plsc.load_gather(mm_v, [chv + 8])
                val = mn + uv * (mx - mn)
                loc = iv - h * HALF
                inb = (loc >= 0) & (loc < HALF)
                locc = jnp.minimum(jnp.maximum(loc, 0), HALF - 1)
                plsc.store_scatter(half_v, [locc], val, mask=inb)
                return carry

            lax.fori_loop(0, IDX_CHUNK // 16, inner, 0)
        pltpu.sync_copy(half_v, out_hbm.at[b, pl.ds(h * HALF, HALF)])


def kernel(x, labels, W, b):
    xf = x.reshape(B_, D_)

    negabs = pl.pallas_call(
        _negabs_grad_body,
        grid=(D_ // DT,),
        in_specs=[
            pl.BlockSpec((B_, NC_), lambda i: (0, 0)),
            pl.BlockSpec((DT, NC_), lambda i: (i, 0)),
        ],
        out_specs=pl.BlockSpec((B_, DT), lambda i: (0, i)),
        out_shape=jax.ShapeDtypeStruct((B_, D_), jnp.float32),
    )(labels, W)

    mm = pl.pallas_call(
        _minmax_body,
        grid=(B_,),
        in_specs=[pl.BlockSpec((1, 1, D_), lambda i: (i, 0, 0))],
        out_specs=pl.BlockSpec((1, 1, 16), lambda i: (i, 0, 0)),
        out_shape=jax.ShapeDtypeStruct((B_, 1, 16), jnp.float32),
    )(xf.reshape(B_, 1, D_)).reshape(B_, 16)

    _, idx = lax.top_k(negabs, K_SEL)
    u = jax.random.uniform(jax.random.key(1), (B_, K_SEL), dtype=jnp.float32)

    mesh = plsc.VectorSubcoreMesh(core_axis_name="c", subcore_axis_name="s")
    scatter_fn = pl.kernel(
        _sc_scatter_body,
        mesh=mesh,
        compiler_params=pltpu.CompilerParams(needs_layout_passes=False),
        out_type=jax.ShapeDtypeStruct((B_, D_), jnp.float32),
        scratch_types=[
            pltpu.VMEM((HALF,), jnp.float32),
            pltpu.VMEM((IDX_CHUNK,), jnp.int32),
            pltpu.VMEM((IDX_CHUNK,), jnp.float32),
            pltpu.VMEM((16,), jnp.float32),
        ],
    )
    out = scatter_fn(xf, idx, u, mm)
    return out.reshape(B_, C_, H_, W_)


# speedup vs baseline: 18.7188x; 4.5742x over previous
import jax
import jax.numpy as jnp
from jax import lax
from jax.experimental import pallas as pl
from jax.experimental.pallas import tpu as pltpu
from jax.experimental.pallas import tpu_sc as plsc

K_SEL = 16384
B_, C_, H_, W_ = 64, 3, 224, 224
CHAN = H_ * W_
D_ = C_ * CHAN
HALF = D_ // 2
NC_ = 100
DT = 1024
IDX_CHUNK = 2048
CCHUNK = 3072
MAXPOS = 0x7F800000


def _absgrad_body(lab_ref, w_ref, o_ref):
    g = lax.dot_general(
        lab_ref[...], w_ref[...], (((1,), (1,)), ((), ())),
        preferred_element_type=jnp.float32)
    o_ref[...] = jnp.abs(g)


def _minmax_body(x_ref, o_ref):
    row = x_ref[...].reshape(1, D_)
    pos = lax.broadcasted_iota(jnp.int32, (1, D_), 1)
    p = pos % CHAN
    ch = pos // CHAN
    valid = p < (CHAN - 1)
    lane = lax.broadcasted_iota(jnp.int32, (1, 16), 1)
    r = jnp.zeros((1, 16), jnp.float32)
    for c in range(C_):
        sel = valid & (ch == c)
        mn = jnp.min(jnp.where(sel, row, jnp.inf))
        mx = jnp.max(jnp.where(sel, row, -jnp.inf))
        r = jnp.where(lane == c, mn, r)
        r = jnp.where(lane == c + 8, mx, r)
    o_ref[...] = r.reshape(1, 1, 16)


RT = 8


def _thresh_body(a_ref, t_ref, tie_ref):
    a = a_ref[...]

    def it(_, carry):
        lo, hi = carry
        mid = lo + lax.div(hi - lo, 2)
        t = lax.bitcast_convert_type(mid, jnp.float32)
        cnt = jnp.sum((a <= t).astype(jnp.int32), axis=1, keepdims=True)
        ge = cnt >= K_SEL
        return jnp.where(ge, lo, mid + 1), jnp.where(ge, mid, hi)

    lo0 = jnp.zeros((RT, 1), jnp.int32)
    hi0 = jnp.full((RT, 1), MAXPOS, jnp.int32)
    lo, hi = lax.fori_loop(0, 31, it, (lo0, hi0))
    tb = lax.bitcast_convert_type(hi, jnp.float32)
    cl = jnp.sum((a < tb).astype(jnp.int32), axis=1, keepdims=True)
    tie = K_SEL - cl
    t_ref[...] = jnp.broadcast_to(tb, (RT, 16)).reshape(RT, 1, 16)
    tie_ref[...] = jnp.broadcast_to(tie, (RT, 16)).reshape(RT, 1, 16)


def _sc_compact_body(a_hbm, t_hbm, tie_hbm, ck_hbm, ci_hbm,
                     chunk_v, ck_v, ci_v, t_v, tie_v):
    cid = lax.axis_index("c")
    sid = lax.axis_index("s")
    wid = sid * 2 + cid
    lane = lax.iota(jnp.int32, 16)
    zero16 = jnp.zeros((16,), jnp.int32)
    for r in range(2):
        b = wid + 32 * r
        pltpu.sync_copy(t_hbm.at[b], t_v)
        pltpu.sync_copy(tie_hbm.at[b], tie_v)
        tv = t_v[...]
        tiev = tie_v[...]
        carry = (zero16, zero16)
        for c in range(D_ // CCHUNK):
            pltpu.sync_copy(a_hbm.at[b, pl.ds(c * CCHUNK, CCHUNK)], chunk_v)

            def vec_loop(j, cr, c=c):
                offl, offe = cr
                a16 = chunk_v[pl.ds(j * 16, 16)]
                idx16 = lane + (c * CCHUNK + j * 16)
                less = a16 < tv
                eq = a16 == tv
                cl_ = plsc.cumsum(less.astype(jnp.int32))
                posl = offl + cl_ - 1
                poslc = jnp.minimum(jnp.maximum(posl, 0), K_SEL - 1)
                plsc.store_scatter(ck_v, [poslc], a16, mask=less)
                plsc.store_scatter(ci_v, [poslc], idx16, mask=less)
                ce_ = plsc.cumsum(eq.astype(jnp.int32))
                erank = offe + ce_ - 1
                sel_e = eq & (erank < tiev)
                pose = K_SEL + erank - tiev
                posec = jnp.minimum(jnp.maximum(pose, 0), K_SEL - 1)
                plsc.store_scatter(ck_v, [posec], a16, mask=sel_e)
                plsc.store_scatter(ci_v, [posec], idx16, mask=sel_e)
                nl = plsc.all_reduce_population_count(less)
                ne = plsc.all_reduce_population_count(eq)
                return offl + nl, offe + ne

            carry = lax.fori_loop(0, CCHUNK // 16, vec_loop, carry)
        pltpu.sync_copy(ck_v, ck_hbm.at[b])
        pltpu.sync_copy(ci_v, ci_hbm.at[b])


RB = 4


def _bitonic_body(k_ref, i_ref, o_ref):
    kv = k_ref[...]
    vv = i_ref[...]
    sh = (RB * 128, 128)
    s_pos = lax.broadcasted_iota(jnp.int32, sh, 0) & 127
    c_io = lax.broadcasted_iota(jnp.int32, sh, 1)
    lin = s_pos * 128 + c_io
    for kk in [2 << t for t in range(14)]:
        jj = kk // 2
        while jj >= 1:
            if jj >= 128:
                m = jj // 128
                sel = (s_pos & m) == 0
                kp = jnp.where(sel, jnp.roll(kv, -m, axis=0),
                               jnp.roll(kv, m, axis=0))
                vp = jnp.where(sel, jnp.roll(vv, -m, axis=0),
                               jnp.roll(vv, m, axis=0))
            else:
                sel = (c_io & jj) == 0
                kp = jnp.where(sel, jnp.roll(kv, -jj, axis=1),
                               jnp.roll(kv, jj, axis=1))
                vp = jnp.where(sel, jnp.roll(vv, -jj, axis=1),
                               jnp.roll(vv, jj, axis=1))
            up = (lin & kk) == 0
            less = (kv < kp) | ((kv == kp) & (vv < vp))
            take_self = less ^ up ^ sel
            kv = jnp.where(take_self, kv, kp)
            vv = jnp.where(take_self, vv, vp)
            jj //= 2
    o_ref[...] = vv


def _sc_scatter_body(x_hbm, idx_hbm, u_hbm, mm_hbm, out_hbm,
                     half_v, idx_v, u_v, mm_v):
    cid = lax.axis_index("c")
    sid = lax.axis_index("s")
    wid = sid * 2 + cid
    for i in range(4):
        h = i // 2
        b = wid + 32 * (i % 2)
        pltpu.sync_copy(x_hbm.at[b, pl.ds(h * HALF, HALF)], half_v)
        pltpu.sync_copy(mm_hbm.at[b], mm_v)
        for cch in range(K_SEL // IDX_CHUNK):
            pltpu.sync_copy(idx_hbm.at[b, pl.ds(cch * IDX_CHUNK, IDX_CHUNK)],
                            idx_v)
            pltpu.sync_copy(u_hbm.at[b, pl.ds(cch * IDX_CHUNK, IDX_CHUNK)],
                            u_v)

            def inner(j, carry, h=h):
                iv = idx_v[pl.ds(j * 16, 16)]
                uv = u_v[pl.ds(j * 16, 16)]
                chv = lax.div(iv, CHAN)
                mn = plsc.load_gather(mm_v, [chv])
                mx = plsc.load_gather(mm_v, [chv + 8])
                val = mn + uv * (mx - mn)
                loc = iv - h * HALF
                inb = (loc >= 0) & (loc < HALF)
                locc = jnp.minimum(jnp.maximum(loc, 0), HALF - 1)
                plsc.store_scatter(half_v, [locc], val, mask=inb)
                return carry

            lax.fori_loop(0, IDX_CHUNK // 16, inner, 0)
        pltpu.sync_copy(half_v, out_hbm.at[b, pl.ds(h * HALF, HALF)])


def kernel(x, labels, W, b):
    xf = x.reshape(B_, D_)
    sc_params = pltpu.CompilerParams(needs_layout_passes=False)
    mesh = plsc.VectorSubcoreMesh(core_axis_name="c", subcore_axis_name="s")

    absg = pl.pallas_call(
        _absgrad_body,
        grid=(D_ // DT,),
        in_specs=[
            pl.BlockSpec((B_, NC_), lambda i: (0, 0)),
            pl.BlockSpec((DT, NC_), lambda i: (i, 0)),
        ],
        out_specs=pl.BlockSpec((B_, DT), lambda i: (0, i)),
        out_shape=jax.ShapeDtypeStruct((B_, D_), jnp.float32),
    )(labels, W)

    mm = pl.pallas_call(
        _minmax_body,
        grid=(B_,),
        in_specs=[pl.BlockSpec((1, 1, D_), lambda i: (i, 0, 0))],
        out_specs=pl.BlockSpec((1, 1, 16), lambda i: (i, 0, 0)),
        out_shape=jax.ShapeDtypeStruct((B_, 1, 16), jnp.float32),
    )(xf.reshape(B_, 1, D_)).reshape(B_, 16)

    tarr, tiearr = pl.pallas_call(
        _thresh_body,
        grid=(B_ // RT,),
        in_specs=[pl.BlockSpec((RT, D_), lambda i: (i, 0))],
        out_specs=(pl.BlockSpec((RT, 1, 16), lambda i: (i, 0, 0)),
                   pl.BlockSpec((RT, 1, 16), lambda i: (i, 0, 0))),
        out_shape=(jax.ShapeDtypeStruct((B_, 1, 16), jnp.float32),
                   jax.ShapeDtypeStruct((B_, 1, 16), jnp.int32)),
    )(absg)
    tarr = tarr.reshape(B_, 16)
    tiearr = tiearr.reshape(B_, 16)

    compact_fn = pl.kernel(
        _sc_compact_body,
        mesh=mesh,
        compiler_params=sc_params,
        out_type=(jax.ShapeDtypeStruct((B_, K_SEL), jnp.float32),
                  jax.ShapeDtypeStruct((B_, K_SEL), jnp.int32)),
        scratch_types=[
            pltpu.VMEM((CCHUNK,), jnp.float32),
            pltpu.VMEM((K_SEL,), jnp.float32),
            pltpu.VMEM((K_SEL,), jnp.int32),
            pltpu.VMEM((16,), jnp.float32),
            pltpu.VMEM((16,), jnp.int32),
        ],
    )
    candk, candi = compact_fn(absg, tarr, tiearr)

    idx = pl.pallas_call(
        _bitonic_body,
        grid=(B_ // RB,),
        in_specs=[pl.BlockSpec((RB * 128, 128), lambda i: (i, 0)),
                  pl.BlockSpec((RB * 128, 128), lambda i: (i, 0))],
        out_specs=pl.BlockSpec((RB * 128, 128), lambda i: (i, 0)),
        out_shape=jax.ShapeDtypeStruct((B_ * 128, 128), jnp.int32),
    )(candk.reshape(B_ * 128, 128), candi.reshape(B_ * 128, 128))
    idx = idx.reshape(B_, K_SEL)

    u = jax.random.uniform(jax.random.key(1), (B_, K_SEL), dtype=jnp.float32)

    scatter_fn = pl.kernel(
        _sc_scatter_body,
        mesh=mesh,
        compiler_params=sc_params,
        out_type=jax.ShapeDtypeStruct((B_, D_), jnp.float32),
        scratch_types=[
            pltpu.VMEM((HALF,), jnp.float32),
            pltpu.VMEM((IDX_CHUNK,), jnp.int32),
            pltpu.VMEM((IDX_CHUNK,), jnp.float32),
            pltpu.VMEM((16,), jnp.float32),
        ],
    )
    out = scatter_fn(xf, idx, u, mm)
    return out.reshape(B_, C_, H_, W_)
